# Initial kernel scaffold; baseline (speedup 1.0000x reference)
import jax
import jax.numpy as jnp
from jax.experimental import pallas as pl


def kernel(x, W_enc, W_dec, b_enc, b_dec, k):
    raise NotImplementedError("write your pallas kernel here")


# speedup vs baseline: 6.4534x; 6.4534x over previous
import functools

import jax
import jax.numpy as jnp
from jax.experimental import pallas as pl
from jax.experimental.pallas import tpu as pltpu


def _encode_body(x_ref, w_ref, b_ref, out_ref):
    out_ref[...] = (
        jnp.dot(x_ref[...], w_ref[0], preferred_element_type=jnp.float32)
        + b_ref[...]
    )


def _select_body(kk_ref, pre_ref, z_ref):
    kk = kk_ref[0]
    v = pre_ref[...]
    bsz = v.shape[0]
    key = jax.lax.bitcast_convert_type(v, jnp.int32)
    skey = jnp.where(key < 0, key ^ jnp.int32(0x7FFFFFFF), key)
    int_min = jnp.int32(-2147483648)

    def bit_step(i, ub):
        b = jnp.int32(31) - i
        cand_u = ub | (jnp.int32(1) << b)
        cand_s = cand_u ^ int_min
        cnt = jnp.sum((skey >= cand_s).astype(jnp.int32), axis=1,
                      keepdims=True)
        return jnp.where(cnt >= kk, cand_u, ub)

    ub = jax.lax.fori_loop(0, 32, bit_step,
                           jnp.zeros((bsz, 1), jnp.int32))
    thresh = ub ^ int_min

    n_gt = jnp.sum((skey > thresh).astype(jnp.int32), axis=1, keepdims=True)
    need = kk - n_gt
    eq = skey == thresh
    idx = jax.lax.broadcasted_iota(jnp.int32, v.shape, 1)

    def idx_step(i, m):
        b = jnp.int32(15) - i
        cand = m + (jnp.int32(1) << b)
        f = jnp.sum((eq & (idx < cand)).astype(jnp.int32), axis=1,
                    keepdims=True)
        return jnp.where(f <= need, cand, m)

    m = jax.lax.fori_loop(0, 16, idx_step, jnp.zeros((bsz, 1), jnp.int32))
    mask = (skey > thresh) | (eq & (idx < m))
    z_ref[...] = jnp.where(mask, jnp.maximum(v, 0.0), 0.0)


def _decode_body(z_ref, w_ref, b_ref, x_ref, xhat_ref, loss_ref, *, n_tiles,
                 inv_btl):
    tl = pl.program_id(0)
    xh = (
        jnp.dot(z_ref[...], w_ref[0], preferred_element_type=jnp.float32)
        + b_ref[0]
    )
    xhat_ref[...] = xh
    d = xh - x_ref[...]
    s = jnp.sum(d * d)

    @pl.when(tl == 0)
    def _init():
        loss_ref[0, 0] = 0.0

    loss_ref[0, 0] += s

    @pl.when(tl == n_tiles - 1)
    def _final():
        loss_ref[0, 0] = loss_ref[0, 0] * inv_btl


def kernel(x, W_enc, W_dec, b_enc, b_dec, k):
    B, T, L, d_in = x.shape
    d_sae = W_enc.shape[-1]
    TL = T * L
    N = TL * d_sae
    K_STATIC = 512

    x2 = x.reshape(B, TL * d_in)
    we = W_enc.reshape(TL, d_in, d_sae)
    wd = W_dec.reshape(TL, d_sae, d_in)
    be = b_enc.reshape(1, d_sae)
    bd = b_dec.reshape(TL, 1, d_in)
    kk = jnp.clip(jnp.asarray(k, jnp.int32), 0, K_STATIC).reshape(1)

    pre = pl.pallas_call(
        _encode_body,
        grid=(TL,),
        in_specs=[
            pl.BlockSpec((B, d_in), lambda i: (0, i)),
            pl.BlockSpec((1, d_in, d_sae), lambda i: (i, 0, 0)),
            pl.BlockSpec((1, d_sae), lambda i: (0, 0)),
        ],
        out_specs=pl.BlockSpec((B, d_sae), lambda i: (0, i)),
        out_shape=jax.ShapeDtypeStruct((B, N), jnp.float32),
    )(x2, we, be)

    z2 = pl.pallas_call(
        _select_body,
        in_specs=[
            pl.BlockSpec(memory_space=pltpu.SMEM),
            pl.BlockSpec((B, N), lambda: (0, 0)),
        ],
        out_specs=pl.BlockSpec((B, N), lambda: (0, 0)),
        out_shape=jax.ShapeDtypeStruct((B, N), jnp.float32),
    )(kk, pre)

    xhat2, loss = pl.pallas_call(
        functools.partial(_decode_body, n_tiles=TL,
                          inv_btl=1.0 / float(B * TL)),
        grid=(TL,),
        in_specs=[
            pl.BlockSpec((B, d_sae), lambda i: (0, i)),
            pl.BlockSpec((1, d_sae, d_in), lambda i: (i, 0, 0)),
            pl.BlockSpec((1, 1, d_in), lambda i: (i, 0, 0)),
            pl.BlockSpec((B, d_in), lambda i: (0, i)),
        ],
        out_specs=[
            pl.BlockSpec((B, d_in), lambda i: (0, i)),
            pl.BlockSpec(memory_space=pltpu.SMEM),
        ],
        out_shape=[
            jax.ShapeDtypeStruct((B, TL * d_in), jnp.float32),
            jax.ShapeDtypeStruct((1, 1), jnp.float32),
        ],
    )(z2, wd, bd, x2)

    x_hat = xhat2.reshape(B, T, L, d_in)
    z = z2.reshape(B, T, L, d_sae)
    return (loss.reshape(()), x_hat, z)
